# baseline (speedup 1.0000x reference)
import jax
import jax.numpy as jnp
from jax.experimental import pallas as pl

_B, _N, _R = 32, 256, 16
_D_IN, _D_H = 128, 128
_BB = 4
_PREC = jax.lax.Precision.DEFAULT


def _rgcn_body(attn_ref, A_ref, We_ref, be_ref,
               Wr0_ref, Lw0_ref, Wr1_ref, Lw1_ref, out_ref):
    f32 = jnp.float32
    for i in range(_BB):
        x = jnp.dot(attn_ref[i], We_ref[...], preferred_element_type=f32, precision=_PREC)
        x = jnp.maximum(x + be_ref[...], 0.0)
        A2 = A_ref[i].reshape(_R * _N, _N).astype(jnp.bfloat16)
        for Wr_ref, Lw_ref in ((Wr0_ref, Lw0_ref), (Wr1_ref, Lw1_ref)):
            nb = jnp.dot(A2, x.astype(jnp.bfloat16), preferred_element_type=f32, precision=_PREC)
            msg = jax.lax.dot_general(
                nb.reshape(_R, _N, _D_H), Wr_ref[...],
                dimension_numbers=(((2,), (1,)), ((0,), (0,))),
                preferred_element_type=f32, precision=_PREC)
            acc = jnp.dot(x, Lw_ref[...], preferred_element_type=f32, precision=_PREC)
            x = jnp.maximum(acc + msg.sum(axis=0), 0.0)
        out_ref[i] = x


def kernel(attn_fts, rel_edges, W_embed, b_embed,
           rel_weight_0, loop_weight_0, rel_weight_1, loop_weight_1):
    B, N, D_IN = attn_fts.shape
    R = rel_edges.shape[1]
    D_H = W_embed.shape[1]
    b2 = b_embed.reshape(1, D_H)
    grid = (B // _BB,)
    return pl.pallas_call(
        _rgcn_body,
        grid=grid,
        in_specs=[
            pl.BlockSpec((_BB, N, D_IN), lambda b: (b, 0, 0)),
            pl.BlockSpec((_BB, R, N, N), lambda b: (b, 0, 0, 0)),
            pl.BlockSpec((D_IN, D_H), lambda b: (0, 0)),
            pl.BlockSpec((1, D_H), lambda b: (0, 0)),
            pl.BlockSpec((R, D_H, D_H), lambda b: (0, 0, 0)),
            pl.BlockSpec((D_H, D_H), lambda b: (0, 0)),
            pl.BlockSpec((R, D_H, D_H), lambda b: (0, 0, 0)),
            pl.BlockSpec((D_H, D_H), lambda b: (0, 0)),
        ],
        out_specs=pl.BlockSpec((_BB, N, D_H), lambda b: (b, 0, 0)),
        out_shape=jax.ShapeDtypeStruct((B, N, D_H), jnp.float32),
    )(attn_fts, rel_edges, W_embed, b2,
      rel_weight_0, loop_weight_0, rel_weight_1, loop_weight_1)


# speedup vs baseline: 1.4473x; 1.4473x over previous
import jax
import jax.numpy as jnp
from jax.experimental import pallas as pl

_B, _N, _R = 32, 256, 16
_D_IN, _D_H = 128, 128
_BB = 2
_PREC = jax.lax.Precision.DEFAULT


def _rgcn_body(attn_ref, A_ref, We_ref, be_ref,
               Wr0_ref, Lw0_ref, Wr1_ref, Lw1_ref, out_ref):
    f32 = jnp.float32
    bf16 = jnp.bfloat16
    We = We_ref[...]
    Wr0 = Wr0_ref[...]
    Lw0 = Lw0_ref[...]
    Wr1 = Wr1_ref[...]
    Lw1 = Lw1_ref[...]
    xs = []
    A2s = []
    for i in range(_BB):
        x = jnp.dot(attn_ref[i], We, preferred_element_type=f32, precision=_PREC)
        xs.append(jnp.maximum(x + be_ref[...], 0.0))
        A2s.append(A_ref[i].reshape(_R * _N, _N))
    for Wr, Lw in ((Wr0, Lw0), (Wr1, Lw1)):
        xbs = xs
        nbs = [jnp.dot(A2s[i], xbs[i], preferred_element_type=f32, precision=_PREC)
               for i in range(_BB)]
        msgs = [jax.lax.dot_general(
                    nbs[i].reshape(_R, _N, _D_H), Wr,
                    dimension_numbers=(((2,), (1,)), ((0,), (0,))),
                    preferred_element_type=f32, precision=_PREC)
                for i in range(_BB)]
        accs = [jnp.dot(xbs[i], Lw, preferred_element_type=f32, precision=_PREC)
                for i in range(_BB)]
        xs = [jnp.maximum(accs[i] + msgs[i].sum(axis=0), 0.0) for i in range(_BB)]
    for i in range(_BB):
        out_ref[i] = xs[i]


def kernel(attn_fts, rel_edges, W_embed, b_embed,
           rel_weight_0, loop_weight_0, rel_weight_1, loop_weight_1):
    B, N, D_IN = attn_fts.shape
    R = rel_edges.shape[1]
    D_H = W_embed.shape[1]
    b2 = b_embed.reshape(1, D_H)
    grid = (B // _BB,)
    return pl.pallas_call(
        _rgcn_body,
        grid=grid,
        in_specs=[
            pl.BlockSpec((_BB, N, D_IN), lambda b: (b, 0, 0)),
            pl.BlockSpec((_BB, R, N, N), lambda b: (b, 0, 0, 0)),
            pl.BlockSpec((D_IN, D_H), lambda b: (0, 0)),
            pl.BlockSpec((1, D_H), lambda b: (0, 0)),
            pl.BlockSpec((R, D_H, D_H), lambda b: (0, 0, 0)),
            pl.BlockSpec((D_H, D_H), lambda b: (0, 0)),
            pl.BlockSpec((R, D_H, D_H), lambda b: (0, 0, 0)),
            pl.BlockSpec((D_H, D_H), lambda b: (0, 0)),
        ],
        out_specs=pl.BlockSpec((_BB, N, D_H), lambda b: (b, 0, 0)),
        out_shape=jax.ShapeDtypeStruct((B, N, D_H), jnp.float32),
    )(attn_fts, rel_edges, W_embed, b2,
      rel_weight_0, loop_weight_0, rel_weight_1, loop_weight_1)


# speedup vs baseline: 1.4951x; 1.0330x over previous
import jax
import jax.numpy as jnp
from jax.experimental import pallas as pl

_B, _N, _R = 32, 256, 16
_D_IN, _D_H = 128, 128
_BB = 4
_PREC = jax.lax.Precision.DEFAULT


def _rgcn_body(attn_ref, A_ref, We_ref, be_ref,
               Wr0_ref, Lw0_ref, Wr1_ref, Lw1_ref, out_ref):
    f32 = jnp.float32
    bf16 = jnp.bfloat16
    We = We_ref[...]
    Wr0 = Wr0_ref[...]
    Lw0 = Lw0_ref[...]
    Wr1 = Wr1_ref[...]
    Lw1 = Lw1_ref[...]
    xs = []
    A2s = []
    for i in range(_BB):
        x = jnp.dot(attn_ref[i], We, preferred_element_type=f32, precision=_PREC)
        xs.append(jnp.maximum(x + be_ref[...], 0.0))
        A2s.append(A_ref[i].reshape(_R * _N, _N))
    for Wr, Lw in ((Wr0, Lw0), (Wr1, Lw1)):
        xbs = xs
        nbs = [jnp.dot(A2s[i], xbs[i], preferred_element_type=f32, precision=_PREC)
               for i in range(_BB)]
        msgs = [jax.lax.dot_general(
                    nbs[i].reshape(_R, _N, _D_H), Wr,
                    dimension_numbers=(((2,), (1,)), ((0,), (0,))),
                    preferred_element_type=f32, precision=_PREC)
                for i in range(_BB)]
        accs = [jnp.dot(xbs[i], Lw, preferred_element_type=f32, precision=_PREC)
                for i in range(_BB)]
        xs = [jnp.maximum(accs[i] + msgs[i].sum(axis=0), 0.0) for i in range(_BB)]
    for i in range(_BB):
        out_ref[i] = xs[i]


def kernel(attn_fts, rel_edges, W_embed, b_embed,
           rel_weight_0, loop_weight_0, rel_weight_1, loop_weight_1):
    B, N, D_IN = attn_fts.shape
    R = rel_edges.shape[1]
    D_H = W_embed.shape[1]
    b2 = b_embed.reshape(1, D_H)
    grid = (B // _BB,)
    return pl.pallas_call(
        _rgcn_body,
        grid=grid,
        in_specs=[
            pl.BlockSpec((_BB, N, D_IN), lambda b: (b, 0, 0)),
            pl.BlockSpec((_BB, R, N, N), lambda b: (b, 0, 0, 0)),
            pl.BlockSpec((D_IN, D_H), lambda b: (0, 0)),
            pl.BlockSpec((1, D_H), lambda b: (0, 0)),
            pl.BlockSpec((R, D_H, D_H), lambda b: (0, 0, 0)),
            pl.BlockSpec((D_H, D_H), lambda b: (0, 0)),
            pl.BlockSpec((R, D_H, D_H), lambda b: (0, 0, 0)),
            pl.BlockSpec((D_H, D_H), lambda b: (0, 0)),
        ],
        out_specs=pl.BlockSpec((_BB, N, D_H), lambda b: (b, 0, 0)),
        out_shape=jax.ShapeDtypeStruct((B, N, D_H), jnp.float32),
    )(attn_fts, rel_edges, W_embed, b2,
      rel_weight_0, loop_weight_0, rel_weight_1, loop_weight_1)


# speedup vs baseline: 1.4957x; 1.0004x over previous
import jax
import jax.numpy as jnp
from jax.experimental import pallas as pl
from jax.experimental.pallas import tpu as pltpu

_B, _N, _R = 32, 256, 16
_D_IN, _D_H = 128, 128
_BB = 4
_PREC = jax.lax.Precision.DEFAULT


def _rgcn_body(attn_ref, A_ref, We_ref, be_ref,
               Wr0_ref, Lw0_ref, Wr1_ref, Lw1_ref, out_ref):
    f32 = jnp.float32
    bf16 = jnp.bfloat16
    We = We_ref[...]
    Wr0 = Wr0_ref[...]
    Lw0 = Lw0_ref[...]
    Wr1 = Wr1_ref[...]
    Lw1 = Lw1_ref[...]
    xs = []
    A2s = []
    for i in range(_BB):
        x = jnp.dot(attn_ref[i], We, preferred_element_type=f32, precision=_PREC)
        xs.append(jnp.maximum(x + be_ref[...], 0.0))
        A2s.append(A_ref[i].reshape(_R * _N, _N))
    for Wr, Lw in ((Wr0, Lw0), (Wr1, Lw1)):
        xbs = xs
        nbs = [jnp.dot(A2s[i], xbs[i], preferred_element_type=f32, precision=_PREC)
               for i in range(_BB)]
        msgs = [jax.lax.dot_general(
                    nbs[i].reshape(_R, _N, _D_H), Wr,
                    dimension_numbers=(((2,), (1,)), ((0,), (0,))),
                    preferred_element_type=f32, precision=_PREC)
                for i in range(_BB)]
        accs = [jnp.dot(xbs[i], Lw, preferred_element_type=f32, precision=_PREC)
                for i in range(_BB)]
        xs = [jnp.maximum(accs[i] + msgs[i].sum(axis=0), 0.0) for i in range(_BB)]
    for i in range(_BB):
        out_ref[i] = xs[i]


def kernel(attn_fts, rel_edges, W_embed, b_embed,
           rel_weight_0, loop_weight_0, rel_weight_1, loop_weight_1):
    B, N, D_IN = attn_fts.shape
    R = rel_edges.shape[1]
    D_H = W_embed.shape[1]
    b2 = b_embed.reshape(1, D_H)
    grid = (B // _BB,)
    return pl.pallas_call(
        _rgcn_body,
        grid=grid,
        in_specs=[
            pl.BlockSpec((_BB, N, D_IN), lambda b: (b, 0, 0)),
            pl.BlockSpec((_BB, R, N, N), lambda b: (b, 0, 0, 0)),
            pl.BlockSpec((D_IN, D_H), lambda b: (0, 0)),
            pl.BlockSpec((1, D_H), lambda b: (0, 0)),
            pl.BlockSpec((R, D_H, D_H), lambda b: (0, 0, 0)),
            pl.BlockSpec((D_H, D_H), lambda b: (0, 0)),
            pl.BlockSpec((R, D_H, D_H), lambda b: (0, 0, 0)),
            pl.BlockSpec((D_H, D_H), lambda b: (0, 0)),
        ],
        out_specs=pl.BlockSpec((_BB, N, D_H), lambda b: (b, 0, 0)),
        out_shape=jax.ShapeDtypeStruct((B, N, D_H), jnp.float32),
        compiler_params=pltpu.CompilerParams(
            dimension_semantics=("parallel",)),
    )(attn_fts, rel_edges, W_embed, b2,
      rel_weight_0, loop_weight_0, rel_weight_1, loop_weight_1)


# speedup vs baseline: 1.5079x; 1.0082x over previous
import jax
import jax.numpy as jnp
from jax.experimental import pallas as pl
from jax.experimental.pallas import tpu as pltpu

_B, _N, _R = 32, 256, 16
_D_IN, _D_H = 128, 128
_BB = 4
_PREC = jax.lax.Precision.DEFAULT


def _rgcn_body(attn_ref, A_ref, We_ref, be_ref,
               Wr0_ref, Lw0_ref, Wr1_ref, Lw1_ref, out_ref):
    f32 = jnp.float32
    bf16 = jnp.bfloat16
    We = We_ref[...].astype(bf16)
    Wr0 = Wr0_ref[...].astype(bf16)
    Lw0 = Lw0_ref[...].astype(bf16)
    Wr1 = Wr1_ref[...].astype(bf16)
    Lw1 = Lw1_ref[...].astype(bf16)
    xs = []
    A2s = []
    for i in range(_BB):
        x = jnp.dot(attn_ref[i].astype(bf16), We, preferred_element_type=f32, precision=_PREC)
        xs.append(jnp.maximum(x + be_ref[...], 0.0))
        A2s.append(A_ref[i].reshape(_R * _N, _N).astype(bf16))
    for Wr, Lw in ((Wr0, Lw0), (Wr1, Lw1)):
        xbs = [x.astype(bf16) for x in xs]
        nbs = [jnp.dot(A2s[i], xbs[i], preferred_element_type=f32, precision=_PREC)
               for i in range(_BB)]
        msgs = [jax.lax.dot_general(
                    nbs[i].reshape(_R, _N, _D_H).astype(bf16), Wr,
                    dimension_numbers=(((2,), (1,)), ((0,), (0,))),
                    preferred_element_type=f32, precision=_PREC)
                for i in range(_BB)]
        accs = [jnp.dot(xbs[i], Lw, preferred_element_type=f32, precision=_PREC)
                for i in range(_BB)]
        xs = [jnp.maximum(accs[i] + msgs[i].sum(axis=0), 0.0) for i in range(_BB)]
    for i in range(_BB):
        out_ref[i] = xs[i]


def kernel(attn_fts, rel_edges, W_embed, b_embed,
           rel_weight_0, loop_weight_0, rel_weight_1, loop_weight_1):
    B, N, D_IN = attn_fts.shape
    R = rel_edges.shape[1]
    D_H = W_embed.shape[1]
    b2 = b_embed.reshape(1, D_H)
    grid = (B // _BB,)
    return pl.pallas_call(
        _rgcn_body,
        grid=grid,
        in_specs=[
            pl.BlockSpec((_BB, N, D_IN), lambda b: (b, 0, 0)),
            pl.BlockSpec((_BB, R, N, N), lambda b: (b, 0, 0, 0)),
            pl.BlockSpec((D_IN, D_H), lambda b: (0, 0)),
            pl.BlockSpec((1, D_H), lambda b: (0, 0)),
            pl.BlockSpec((R, D_H, D_H), lambda b: (0, 0, 0)),
            pl.BlockSpec((D_H, D_H), lambda b: (0, 0)),
            pl.BlockSpec((R, D_H, D_H), lambda b: (0, 0, 0)),
            pl.BlockSpec((D_H, D_H), lambda b: (0, 0)),
        ],
        out_specs=pl.BlockSpec((_BB, N, D_H), lambda b: (b, 0, 0)),
        out_shape=jax.ShapeDtypeStruct((B, N, D_H), jnp.float32),
        compiler_params=pltpu.CompilerParams(
            dimension_semantics=("parallel",)),
    )(attn_fts, rel_edges, W_embed, b2,
      rel_weight_0, loop_weight_0, rel_weight_1, loop_weight_1)
